# Initial kernel scaffold; baseline (speedup 1.0000x reference)
import jax
import jax.numpy as jnp
from jax.experimental import pallas as pl


def kernel(input_ids, table):
    raise NotImplementedError("write your pallas kernel here")


# speedup vs baseline: 1.6845x; 1.6845x over previous
import functools

import jax
import jax.numpy as jnp
from jax import lax
from jax.experimental import pallas as pl
from jax.experimental.pallas import tpu as pltpu
from jax.experimental.pallas import tpu_sc as plsc

D = 64
CHUNK = 128

_info = plsc.get_sparse_core_info()
_NC = _info.num_cores
_NS = _info.num_subcores
_NW = _NC * _NS


@functools.lru_cache(maxsize=None)
def _build(n_total):
    n_per_w = n_total // _NW
    n_chunks = n_per_w // CHUNK
    mesh = plsc.VectorSubcoreMesh(core_axis_name="c", subcore_axis_name="s")

    @functools.partial(
        pl.kernel,
        mesh=mesh,
        compiler_params=pltpu.CompilerParams(use_tc_tiling_on_sc=False),
        out_type=jax.ShapeDtypeStruct((n_total, D), jnp.float32),
        scratch_types=[
            pltpu.VMEM((n_chunks, CHUNK), jnp.int32),
            pltpu.VMEM((CHUNK, D), jnp.float32),
            pltpu.SemaphoreType.DMA,
        ],
    )
    def emb(idx_hbm, table_hbm, out_hbm, idx_v, rows, gsem):
        wid = lax.axis_index("s") * _NC + lax.axis_index("c")
        base = wid * n_per_w
        pltpu.sync_copy(idx_hbm.at[pl.ds(wid * n_chunks, n_chunks)], idx_v)

        def body(j, carry):
            pltpu.async_copy(table_hbm.at[idx_v.at[j]], rows, gsem).wait()
            pltpu.sync_copy(rows, out_hbm.at[pl.ds(base + j * CHUNK, CHUNK)])
            return carry

        lax.fori_loop(0, n_chunks, body, 0)

    return emb


def kernel(input_ids, table):
    b, s = input_ids.shape
    n_total = b * s
    ids = input_ids.reshape(n_total // CHUNK, CHUNK)
    out = _build(n_total)(ids, table)
    return out.reshape(b, s, D)


# speedup vs baseline: 1.8699x; 1.1101x over previous
import functools

import jax
import jax.numpy as jnp
from jax import lax
from jax.experimental import pallas as pl
from jax.experimental.pallas import tpu as pltpu
from jax.experimental.pallas import tpu_sc as plsc

D = 64
CHUNK = 128
K = 4
SLAB = K * CHUNK

_info = plsc.get_sparse_core_info()
_NC = _info.num_cores
_NS = _info.num_subcores
_NW = _NC * _NS


@functools.lru_cache(maxsize=None)
def _build(n_total):
    n_per_w = n_total // _NW
    n_chunks = n_per_w // CHUNK
    n_slabs = n_chunks // K
    assert n_slabs % 2 == 0 and n_slabs >= 2
    mesh = plsc.VectorSubcoreMesh(core_axis_name="c", subcore_axis_name="s")

    @functools.partial(
        pl.kernel,
        mesh=mesh,
        compiler_params=pltpu.CompilerParams(use_tc_tiling_on_sc=False),
        out_type=jax.ShapeDtypeStruct((n_total, D), jnp.float32),
        scratch_types=[
            pltpu.VMEM((n_chunks, CHUNK), jnp.int32),
            pltpu.VMEM((SLAB, D), jnp.float32),
            pltpu.VMEM((SLAB, D), jnp.float32),
            pltpu.SemaphoreType.DMA,
            pltpu.SemaphoreType.DMA,
            pltpu.SemaphoreType.DMA,
            pltpu.SemaphoreType.DMA,
        ],
    )
    def emb(idx_hbm, table_hbm, out_hbm, idx_v, buf_a, buf_b,
            gsem_a, gsem_b, wsem_a, wsem_b):
        wid = lax.axis_index("s") * _NC + lax.axis_index("c")
        base = wid * n_per_w
        pltpu.sync_copy(idx_hbm.at[pl.ds(wid * n_chunks, n_chunks)], idx_v)

        def fire_gathers(slab, buf, sem):
            for t in range(K):
                pltpu.async_copy(
                    table_hbm.at[idx_v.at[slab * K + t]],
                    buf.at[pl.ds(t * CHUNK, CHUNK)],
                    sem)

        def wait_gathers(buf, sem):
            pltpu.make_async_copy(out_hbm.at[pl.ds(0, SLAB)], buf, sem).wait()

        def fire_write(slab, buf, sem):
            return pltpu.async_copy(
                buf, out_hbm.at[pl.ds(base + slab * SLAB, SLAB)], sem)

        def wait_write(buf, sem):
            pltpu.make_async_copy(
                buf, out_hbm.at[pl.ds(base, SLAB)], sem).wait()

        fire_gathers(0, buf_a, gsem_a)

        @pl.loop(0, n_slabs, step=2)
        def body(ja):
            jb = ja + 1
            wait_gathers(buf_a, gsem_a)

            @pl.when(ja > 0)
            def _():
                wait_write(buf_b, wsem_b)

            fire_gathers(jb, buf_b, gsem_b)
            hw_a = fire_write(ja, buf_a, wsem_a)
            wait_gathers(buf_b, gsem_b)
            hw_a.wait()

            @pl.when(jb + 1 < n_slabs)
            def _():
                fire_gathers(ja + 2, buf_a, gsem_a)

            fire_write(jb, buf_b, wsem_b)

        wait_write(buf_b, wsem_b)

    return emb


def kernel(input_ids, table):
    b, s = input_ids.shape
    n_total = b * s
    ids = input_ids.reshape(n_total // CHUNK, CHUNK)
    out = _build(n_total)(ids, table)
    return out.reshape(b, s, D)
